# baseline (speedup 1.0000x reference)
import jax
import jax.numpy as jnp
from jax.experimental import pallas as pl
from jax.experimental.pallas import tpu as pltpu

_E = 8
_TOP_K = 2


def _moe_block_kernel(x_ref, d_ref, gw_ref, gb_ref, ew_ref, eb_ref, out_ref):
    logits = (
        jax.lax.dot_general(
            x_ref[...],
            gw_ref[...],
            (((1,), (0,)), ((), ())),
            preferred_element_type=jnp.float32,
        )
        + gb_ref[...]
    )

    e_iota = jax.lax.broadcasted_iota(jnp.int32, logits.shape, 1)
    m1 = jnp.max(logits, axis=1, keepdims=True)
    i1 = jnp.min(jnp.where(logits == m1, e_iota, _E), axis=1, keepdims=True)
    masked = jnp.where(e_iota == i1, -jnp.inf, logits)
    m2 = jnp.max(masked, axis=1, keepdims=True)
    i2 = jnp.min(jnp.where(masked == m2, e_iota, _E), axis=1, keepdims=True)
    t = jnp.exp(m2 - m1)
    w1 = 1.0 / (1.0 + t)
    w2 = 1.0 - w1
    w = jnp.where(e_iota == i1, w1, 0.0) + jnp.where(e_iota == i2, w2, 0.0)

    d = d_ref[...]
    acc = jax.lax.dot_general(
        w,
        eb_ref[...],
        (((1,), (0,)), ((), ())),
        preferred_element_type=jnp.float32,
    )
    for e in range(_E):
        y = jax.lax.dot_general(
            d,
            ew_ref[e],
            (((1,), (0,)), ((), ())),
            preferred_element_type=jnp.float32,
        )
        acc = acc + w[:, e][:, None] * y
    out_ref[...] = acc


@jax.jit
def kernel(input_feat, delta, gate_W, gate_b, expert_W, expert_b):
    B, T, D = input_feat.shape
    E = expert_W.shape[0]
    N = B * T
    TB = 1024

    x = input_feat.reshape(N, D)
    d = delta.reshape(N, D)
    gb = gate_b.reshape(1, E)

    grid = (N // TB,)
    out = pl.pallas_call(
        _moe_block_kernel,
        grid=grid,
        in_specs=[
            pl.BlockSpec((TB, D), lambda i: (i, 0)),
            pl.BlockSpec((TB, D), lambda i: (i, 0)),
            pl.BlockSpec((D, E), lambda i: (0, 0)),
            pl.BlockSpec((1, E), lambda i: (0, 0)),
            pl.BlockSpec((E, D, D), lambda i: (0, 0, 0)),
            pl.BlockSpec((E, D), lambda i: (0, 0)),
        ],
        out_specs=pl.BlockSpec((TB, D), lambda i: (i, 0)),
        out_shape=jax.ShapeDtypeStruct((N, D), jnp.float32),
        compiler_params=pltpu.CompilerParams(
            dimension_semantics=("arbitrary",),
        ),
    )(x, d, gate_W, gb, expert_W, expert_b)
    return out.reshape(B, T, D)


# speedup vs baseline: 1.0230x; 1.0230x over previous
import jax
import jax.numpy as jnp
from jax.experimental import pallas as pl
from jax.experimental.pallas import tpu as pltpu

_E = 8
_TOP_K = 2


def _moe_block_kernel(x_ref, d_ref, gw_ref, gb_ref, ew_ref, eb_ref, out_ref):
    logits = (
        jax.lax.dot_general(
            x_ref[...],
            gw_ref[...],
            (((1,), (0,)), ((), ())),
            preferred_element_type=jnp.float32,
        )
        + gb_ref[...]
    )

    e_iota = jax.lax.broadcasted_iota(jnp.int32, logits.shape, 1)
    m1 = jnp.max(logits, axis=1, keepdims=True)
    i1 = jnp.min(jnp.where(logits == m1, e_iota, _E), axis=1, keepdims=True)
    masked = jnp.where(e_iota == i1, -jnp.inf, logits)
    m2 = jnp.max(masked, axis=1, keepdims=True)
    i2 = jnp.min(jnp.where(masked == m2, e_iota, _E), axis=1, keepdims=True)
    t = jnp.exp(m2 - m1)
    w1 = 1.0 / (1.0 + t)
    w2 = 1.0 - w1
    w = jnp.where(e_iota == i1, w1, 0.0) + jnp.where(e_iota == i2, w2, 0.0)

    d = d_ref[...]
    acc = jnp.zeros(out_ref.shape, jnp.float32)
    for e in range(_E):
        y = jax.lax.dot_general(
            d,
            ew_ref[e],
            (((1,), (0,)), ((), ())),
            preferred_element_type=jnp.float32,
        )
        y = y + eb_ref[e][None, :]
        acc = acc + w[:, e][:, None] * y
    out_ref[...] = acc


@jax.jit
def kernel(input_feat, delta, gate_W, gate_b, expert_W, expert_b):
    B, T, D = input_feat.shape
    E = expert_W.shape[0]
    N = B * T
    TB = 1024

    x = input_feat.reshape(N, D)
    d = delta.reshape(N, D)
    gb = gate_b.reshape(1, E)

    grid = (N // TB,)
    out = pl.pallas_call(
        _moe_block_kernel,
        grid=grid,
        in_specs=[
            pl.BlockSpec((TB, D), lambda i: (i, 0)),
            pl.BlockSpec((TB, D), lambda i: (i, 0)),
            pl.BlockSpec((D, E), lambda i: (0, 0)),
            pl.BlockSpec((1, E), lambda i: (0, 0)),
            pl.BlockSpec((E, D, D), lambda i: (0, 0, 0)),
            pl.BlockSpec((E, D), lambda i: (0, 0)),
        ],
        out_specs=pl.BlockSpec((TB, D), lambda i: (i, 0)),
        out_shape=jax.ShapeDtypeStruct((N, D), jnp.float32),
        compiler_params=pltpu.CompilerParams(
            dimension_semantics=("arbitrary",),
        ),
    )(x, d, gate_W, gb, expert_W, expert_b)
    return out.reshape(B, T, D)
